# baseline (speedup 1.0000x reference)
import numpy as np
import jax
import jax.numpy as jnp
from jax.experimental import pallas as pl
from jax.experimental.pallas import tpu as pltpu

_DT_RANK = 160
_D_STATE = 32

_DBLK1 = 512
_DBLK2 = 256
_L2 = _DBLK2 * _D_STATE

_E = np.repeat(np.eye(_DBLK2, dtype=np.float32), _D_STATE, axis=1)
_T = np.tile(np.eye(_D_STATE, dtype=np.float32), (1, _DBLK2))
_R = np.ascontiguousarray(_E.T)


def _stage1_body(x_ref, wssm_ref, wmlp_ref, cs_ref, cw_ref, cb_ref, xp_ref,
                 u_ref, res_ref, dbl_ref):
    i = pl.program_id(1)
    x = x_ref[...]
    xssm = jnp.dot(x, wssm_ref[...], preferred_element_type=jnp.float32)
    res = jax.nn.silu(jnp.dot(x, wmlp_ref[...],
                              preferred_element_type=jnp.float32))
    cw = cw_ref[...]
    conv = (cs_ref[1] * cw[0:1] + cs_ref[2] * cw[1:2] + cs_ref[3] * cw[2:3]
            + xssm * cw[3:4] + cb_ref[...])
    u = jax.nn.silu(conv)
    u_ref[...] = u
    res_ref[...] = res
    contrib = jnp.dot(u, xp_ref[...], preferred_element_type=jnp.float32)

    @pl.when(i == 0)
    def _():
        dbl_ref[0] = contrib

    @pl.when(i != 0)
    def _():
        dbl_ref[0] += contrib


def _stage2_body(dtin_ref, bm_ref, cm_ref, dtw_ref, dtb_ref, alog_ref,
                 ssm_ref, u_ref, res_ref, d_ref, outw_ref, e_ref, t_ref,
                 r_ref, out_ref):
    i = pl.program_id(1)
    dt = jax.nn.softplus(
        jnp.dot(dtin_ref[...], dtw_ref[...],
                preferred_element_type=jnp.float32) + dtb_ref[...])
    a_flat = -jnp.exp(alog_ref[...])
    dte = jnp.dot(dt, e_ref[...], preferred_element_type=jnp.float32)
    dA = jnp.exp(dte * a_flat)
    cme = jnp.dot(cm_ref[...], t_ref[...], preferred_element_type=jnp.float32)
    z = dA * ssm_ref[...] * cme
    y1 = jnp.dot(z, r_ref[...], preferred_element_type=jnp.float32)
    bc = jnp.sum(bm_ref[...] * cm_ref[...], axis=1, keepdims=True)
    u = u_ref[...]
    y = y1 + dt * u * bc + d_ref[...] * u
    g = y * res_ref[...]
    contrib = jnp.dot(g, outw_ref[...], preferred_element_type=jnp.float32)

    @pl.when(i == 0)
    def _():
        out_ref[0] = contrib

    @pl.when(i != 0)
    def _():
        out_ref[0] += contrib


def kernel(x, conv_states, ssm_state, w_in_ssm, w_in_mlp, conv_w, conv_b,
           A_log, x_proj_w, dt_proj_w, dt_proj_b, D, out_proj_w):
    B, DM = x.shape[2], x.shape[3]
    DI = w_in_ssm.shape[1]
    x2 = x.reshape(B, DM)
    cb = conv_b.reshape(1, DI)

    n1 = DI // _DBLK1 // 2
    u, res, dbl_parts = pl.pallas_call(
        _stage1_body,
        grid=(2, n1),
        in_specs=[
            pl.BlockSpec((B, DM), lambda c, i: (0, 0)),
            pl.BlockSpec((DM, _DBLK1), lambda c, i: (0, c * n1 + i)),
            pl.BlockSpec((DM, _DBLK1), lambda c, i: (0, c * n1 + i)),
            pl.BlockSpec((4, B, _DBLK1), lambda c, i: (0, 0, c * n1 + i)),
            pl.BlockSpec((4, _DBLK1), lambda c, i: (0, c * n1 + i)),
            pl.BlockSpec((1, _DBLK1), lambda c, i: (0, c * n1 + i)),
            pl.BlockSpec((_DBLK1, _DT_RANK + 2 * _D_STATE),
                         lambda c, i: (c * n1 + i, 0)),
        ],
        out_specs=[
            pl.BlockSpec((B, _DBLK1), lambda c, i: (0, c * n1 + i)),
            pl.BlockSpec((B, _DBLK1), lambda c, i: (0, c * n1 + i)),
            pl.BlockSpec((1, B, _DT_RANK + 2 * _D_STATE),
                         lambda c, i: (c, 0, 0)),
        ],
        out_shape=[
            jax.ShapeDtypeStruct((B, DI), jnp.float32),
            jax.ShapeDtypeStruct((B, DI), jnp.float32),
            jax.ShapeDtypeStruct((2, B, _DT_RANK + 2 * _D_STATE),
                                 jnp.float32),
        ],
        compiler_params=pltpu.CompilerParams(
            dimension_semantics=("parallel", "arbitrary"),
            vmem_limit_bytes=56 * 1024 * 1024,
        ),
    )(x2, w_in_ssm, w_in_mlp, conv_states, conv_w, cb, x_proj_w)

    dbl = dbl_parts[0] + dbl_parts[1]
    dtin = dbl[:, :_DT_RANK]
    bm = dbl[:, _DT_RANK:_DT_RANK + _D_STATE]
    cm = dbl[:, _DT_RANK + _D_STATE:]

    alog_flat = A_log.reshape(1, DI * _D_STATE)
    ssm_flat = ssm_state.reshape(B, DI * _D_STATE)
    dtb = dt_proj_b.reshape(1, DI)
    d2 = D.reshape(1, DI)

    n2 = DI // _DBLK2 // 2
    out_parts = pl.pallas_call(
        _stage2_body,
        grid=(2, n2),
        in_specs=[
            pl.BlockSpec((B, _DT_RANK), lambda c, i: (0, 0)),
            pl.BlockSpec((B, _D_STATE), lambda c, i: (0, 0)),
            pl.BlockSpec((B, _D_STATE), lambda c, i: (0, 0)),
            pl.BlockSpec((_DT_RANK, _DBLK2), lambda c, i: (0, c * n2 + i)),
            pl.BlockSpec((1, _DBLK2), lambda c, i: (0, c * n2 + i)),
            pl.BlockSpec((1, _L2), lambda c, i: (0, c * n2 + i)),
            pl.BlockSpec((B, _L2), lambda c, i: (0, c * n2 + i)),
            pl.BlockSpec((B, _DBLK2), lambda c, i: (0, c * n2 + i)),
            pl.BlockSpec((B, _DBLK2), lambda c, i: (0, c * n2 + i)),
            pl.BlockSpec((1, _DBLK2), lambda c, i: (0, c * n2 + i)),
            pl.BlockSpec((_DBLK2, DM), lambda c, i: (c * n2 + i, 0)),
            pl.BlockSpec((_DBLK2, _L2), lambda c, i: (0, 0)),
            pl.BlockSpec((_D_STATE, _L2), lambda c, i: (0, 0)),
            pl.BlockSpec((_L2, _DBLK2), lambda c, i: (0, 0)),
        ],
        out_specs=pl.BlockSpec((1, B, DM), lambda c, i: (c, 0, 0)),
        out_shape=jax.ShapeDtypeStruct((2, B, DM), jnp.float32),
        compiler_params=pltpu.CompilerParams(
            dimension_semantics=("parallel", "arbitrary"),
            vmem_limit_bytes=56 * 1024 * 1024,
        ),
    )(dtin, bm, cm, dt_proj_w, dtb, alog_flat, ssm_flat, u, res, d2,
      out_proj_w, _E, _T, _R)

    out = out_parts[0] + out_parts[1]
    return out.reshape(1, 1, B, DM)


# speedup vs baseline: 2.5503x; 2.5503x over previous
import jax
import jax.numpy as jnp
from jax.experimental import pallas as pl
from jax.experimental.pallas import tpu as pltpu

_DT_RANK = 160
_D_STATE = 32

_DBLK1 = 512
_DBLK2 = 256


def _stage1_body(x_ref, wssm_ref, wmlp_ref, cs_ref, cw_ref, cb_ref, xpt_ref,
                 u_ref, res_ref, dbl_ref):
    i = pl.program_id(1)
    x = x_ref[...]
    xssm = jnp.dot(x, wssm_ref[...], preferred_element_type=jnp.float32)
    res = jax.nn.silu(jnp.dot(x, wmlp_ref[...],
                              preferred_element_type=jnp.float32))
    cw = cw_ref[...]
    conv = (cs_ref[1] * cw[0:1] + cs_ref[2] * cw[1:2] + cs_ref[3] * cw[2:3]
            + xssm * cw[3:4] + cb_ref[...])
    u = jax.nn.silu(conv)
    u_ref[...] = u
    res_ref[...] = res
    contrib = jax.lax.dot_general(
        u, xpt_ref[...], (((1,), (1,)), ((), ())),
        preferred_element_type=jnp.float32)

    @pl.when(i == 0)
    def _():
        dbl_ref[0] = contrib

    @pl.when(i != 0)
    def _():
        dbl_ref[0] += contrib


def _stage2_body(dtin_ref, bm_ref, cm_ref, dtw_ref, dtb_ref, alogt_ref,
                 ssmt_ref, u_ref, res_ref, d_ref, outw_ref, out_ref):
    i = pl.program_id(1)
    dt = jax.nn.softplus(
        jnp.dot(dtin_ref[...], dtw_ref[...],
                preferred_element_type=jnp.float32) + dtb_ref[...])
    a = -jnp.exp(alogt_ref[...])
    dA = jnp.exp(dt[:, None, :] * a[None, :, :])
    cm = cm_ref[...]
    z = dA * ssmt_ref[...] * cm[:, :, None]
    y1 = jnp.sum(z, axis=1)
    bc = jnp.sum(bm_ref[...] * cm, axis=1, keepdims=True)
    u = u_ref[...]
    y = y1 + dt * u * bc + d_ref[...] * u
    g = y * res_ref[...]
    contrib = jnp.dot(g, outw_ref[...], preferred_element_type=jnp.float32)

    @pl.when(i == 0)
    def _():
        out_ref[0] = contrib

    @pl.when(i != 0)
    def _():
        out_ref[0] += contrib


def kernel(x, conv_states, ssm_state, w_in_ssm, w_in_mlp, conv_w, conv_b,
           A_log, x_proj_w, dt_proj_w, dt_proj_b, D, out_proj_w):
    B, DM = x.shape[2], x.shape[3]
    DI = w_in_ssm.shape[1]
    x2 = x.reshape(B, DM)
    cb = conv_b.reshape(1, DI)
    xpt = x_proj_w.T

    n1 = DI // _DBLK1 // 2
    u, res, dbl_parts = pl.pallas_call(
        _stage1_body,
        grid=(2, n1),
        in_specs=[
            pl.BlockSpec((B, DM), lambda c, i: (0, 0)),
            pl.BlockSpec((DM, _DBLK1), lambda c, i: (0, c * n1 + i)),
            pl.BlockSpec((DM, _DBLK1), lambda c, i: (0, c * n1 + i)),
            pl.BlockSpec((4, B, _DBLK1), lambda c, i: (0, 0, c * n1 + i)),
            pl.BlockSpec((4, _DBLK1), lambda c, i: (0, c * n1 + i)),
            pl.BlockSpec((1, _DBLK1), lambda c, i: (0, c * n1 + i)),
            pl.BlockSpec((_DT_RANK + 2 * _D_STATE, _DBLK1),
                         lambda c, i: (0, c * n1 + i)),
        ],
        out_specs=[
            pl.BlockSpec((B, _DBLK1), lambda c, i: (0, c * n1 + i)),
            pl.BlockSpec((B, _DBLK1), lambda c, i: (0, c * n1 + i)),
            pl.BlockSpec((1, B, _DT_RANK + 2 * _D_STATE),
                         lambda c, i: (c, 0, 0)),
        ],
        out_shape=[
            jax.ShapeDtypeStruct((B, DI), jnp.float32),
            jax.ShapeDtypeStruct((B, DI), jnp.float32),
            jax.ShapeDtypeStruct((2, B, _DT_RANK + 2 * _D_STATE),
                                 jnp.float32),
        ],
        compiler_params=pltpu.CompilerParams(
            dimension_semantics=("parallel", "arbitrary"),
            vmem_limit_bytes=56 * 1024 * 1024,
        ),
    )(x2, w_in_ssm, w_in_mlp, conv_states, conv_w, cb, xpt)

    dbl = dbl_parts[0] + dbl_parts[1]
    dtin = dbl[:, :_DT_RANK]
    bm = dbl[:, _DT_RANK:_DT_RANK + _D_STATE]
    cm = dbl[:, _DT_RANK + _D_STATE:]

    ssmt = ssm_state.transpose(0, 2, 1)
    alogt = A_log.T
    dtb = dt_proj_b.reshape(1, DI)
    d2 = D.reshape(1, DI)

    n2 = DI // _DBLK2 // 2
    out_parts = pl.pallas_call(
        _stage2_body,
        grid=(2, n2),
        in_specs=[
            pl.BlockSpec((B, _DT_RANK), lambda c, i: (0, 0)),
            pl.BlockSpec((B, _D_STATE), lambda c, i: (0, 0)),
            pl.BlockSpec((B, _D_STATE), lambda c, i: (0, 0)),
            pl.BlockSpec((_DT_RANK, _DBLK2), lambda c, i: (0, c * n2 + i)),
            pl.BlockSpec((1, _DBLK2), lambda c, i: (0, c * n2 + i)),
            pl.BlockSpec((_D_STATE, _DBLK2), lambda c, i: (0, c * n2 + i)),
            pl.BlockSpec((B, _D_STATE, _DBLK2),
                         lambda c, i: (0, 0, c * n2 + i)),
            pl.BlockSpec((B, _DBLK2), lambda c, i: (0, c * n2 + i)),
            pl.BlockSpec((B, _DBLK2), lambda c, i: (0, c * n2 + i)),
            pl.BlockSpec((1, _DBLK2), lambda c, i: (0, c * n2 + i)),
            pl.BlockSpec((_DBLK2, DM), lambda c, i: (c * n2 + i, 0)),
        ],
        out_specs=pl.BlockSpec((1, B, DM), lambda c, i: (c, 0, 0)),
        out_shape=jax.ShapeDtypeStruct((2, B, DM), jnp.float32),
        compiler_params=pltpu.CompilerParams(
            dimension_semantics=("parallel", "arbitrary"),
            vmem_limit_bytes=56 * 1024 * 1024,
        ),
    )(dtin, bm, cm, dt_proj_w, dtb, alogt, ssmt, u, res, d2, out_proj_w)

    out = out_parts[0] + out_parts[1]
    return out.reshape(1, 1, B, DM)


# speedup vs baseline: 2.7691x; 1.0858x over previous
import jax
import jax.numpy as jnp
from jax.experimental import pallas as pl
from jax.experimental.pallas import tpu as pltpu

_DT_RANK = 160
_D_STATE = 32

_DBLK1 = 512
_DBLK2 = 512


def _stage1_body(x_ref, wssm_ref, wmlp_ref, cs_ref, cw_ref, cb_ref, xpt_ref,
                 u_ref, res_ref, dbl_ref):
    i = pl.program_id(1)
    x = x_ref[...]
    xssm = jnp.dot(x, wssm_ref[...], preferred_element_type=jnp.float32)
    res = jax.nn.silu(jnp.dot(x, wmlp_ref[...],
                              preferred_element_type=jnp.float32))
    cw = cw_ref[...]
    conv = (cs_ref[1] * cw[0:1] + cs_ref[2] * cw[1:2] + cs_ref[3] * cw[2:3]
            + xssm * cw[3:4] + cb_ref[...])
    u = jax.nn.silu(conv)
    u_ref[...] = u
    res_ref[...] = res
    contrib = jax.lax.dot_general(
        u, xpt_ref[...], (((1,), (1,)), ((), ())),
        preferred_element_type=jnp.float32)

    @pl.when(i == 0)
    def _():
        dbl_ref[0] = contrib

    @pl.when(i != 0)
    def _():
        dbl_ref[0] += contrib


def _stage2_body(dblp_ref, dtw_ref, dtb_ref, alogt_ref,
                 ssmt_ref, u_ref, res_ref, d_ref, outw_ref, out_ref):
    i = pl.program_id(1)
    dbl = dblp_ref[0] + dblp_ref[1]
    dtin = dbl[:, :_DT_RANK]
    bm = dbl[:, _DT_RANK:_DT_RANK + _D_STATE]
    cm = dbl[:, _DT_RANK + _D_STATE:]
    dt = jax.nn.softplus(
        jnp.dot(dtin, dtw_ref[...],
                preferred_element_type=jnp.float32) + dtb_ref[...])
    a = -jnp.exp(alogt_ref[...])
    dA = jnp.exp(dt[:, None, :] * a[None, :, :])
    z = dA * ssmt_ref[...] * cm[:, :, None]
    y1 = jnp.sum(z, axis=1)
    bc = jnp.sum(bm * cm, axis=1, keepdims=True)
    u = u_ref[...]
    y = y1 + dt * u * bc + d_ref[...] * u
    g = y * res_ref[...]
    contrib = jnp.dot(g, outw_ref[...], preferred_element_type=jnp.float32)

    @pl.when(i == 0)
    def _():
        out_ref[0] = contrib

    @pl.when(i != 0)
    def _():
        out_ref[0] += contrib


def kernel(x, conv_states, ssm_state, w_in_ssm, w_in_mlp, conv_w, conv_b,
           A_log, x_proj_w, dt_proj_w, dt_proj_b, D, out_proj_w):
    B, DM = x.shape[2], x.shape[3]
    DI = w_in_ssm.shape[1]
    x2 = x.reshape(B, DM)
    cb = conv_b.reshape(1, DI)
    xpt = x_proj_w.T

    n1 = DI // _DBLK1 // 2
    u, res, dbl_parts = pl.pallas_call(
        _stage1_body,
        grid=(2, n1),
        in_specs=[
            pl.BlockSpec((B, DM), lambda c, i: (0, 0)),
            pl.BlockSpec((DM, _DBLK1), lambda c, i: (0, c * n1 + i)),
            pl.BlockSpec((DM, _DBLK1), lambda c, i: (0, c * n1 + i)),
            pl.BlockSpec((4, B, _DBLK1), lambda c, i: (0, 0, c * n1 + i)),
            pl.BlockSpec((4, _DBLK1), lambda c, i: (0, c * n1 + i)),
            pl.BlockSpec((1, _DBLK1), lambda c, i: (0, c * n1 + i)),
            pl.BlockSpec((_DT_RANK + 2 * _D_STATE, _DBLK1),
                         lambda c, i: (0, c * n1 + i)),
        ],
        out_specs=[
            pl.BlockSpec((B, _DBLK1), lambda c, i: (0, c * n1 + i)),
            pl.BlockSpec((B, _DBLK1), lambda c, i: (0, c * n1 + i)),
            pl.BlockSpec((1, B, _DT_RANK + 2 * _D_STATE),
                         lambda c, i: (c, 0, 0)),
        ],
        out_shape=[
            jax.ShapeDtypeStruct((B, DI), jnp.float32),
            jax.ShapeDtypeStruct((B, DI), jnp.float32),
            jax.ShapeDtypeStruct((2, B, _DT_RANK + 2 * _D_STATE),
                                 jnp.float32),
        ],
        compiler_params=pltpu.CompilerParams(
            dimension_semantics=("parallel", "arbitrary"),
            vmem_limit_bytes=56 * 1024 * 1024,
        ),
    )(x2, w_in_ssm, w_in_mlp, conv_states, conv_w, cb, xpt)

    ssmt = ssm_state.transpose(0, 2, 1)
    alogt = A_log.T
    dtb = dt_proj_b.reshape(1, DI)
    d2 = D.reshape(1, DI)

    n2 = DI // _DBLK2 // 2
    out_parts = pl.pallas_call(
        _stage2_body,
        grid=(2, n2),
        in_specs=[
            pl.BlockSpec((2, B, _DT_RANK + 2 * _D_STATE),
                         lambda c, i: (0, 0, 0)),
            pl.BlockSpec((_DT_RANK, _DBLK2), lambda c, i: (0, c * n2 + i)),
            pl.BlockSpec((1, _DBLK2), lambda c, i: (0, c * n2 + i)),
            pl.BlockSpec((_D_STATE, _DBLK2), lambda c, i: (0, c * n2 + i)),
            pl.BlockSpec((B, _D_STATE, _DBLK2),
                         lambda c, i: (0, 0, c * n2 + i)),
            pl.BlockSpec((B, _DBLK2), lambda c, i: (0, c * n2 + i)),
            pl.BlockSpec((B, _DBLK2), lambda c, i: (0, c * n2 + i)),
            pl.BlockSpec((1, _DBLK2), lambda c, i: (0, c * n2 + i)),
            pl.BlockSpec((_DBLK2, DM), lambda c, i: (c * n2 + i, 0)),
        ],
        out_specs=pl.BlockSpec((1, B, DM), lambda c, i: (c, 0, 0)),
        out_shape=jax.ShapeDtypeStruct((2, B, DM), jnp.float32),
        compiler_params=pltpu.CompilerParams(
            dimension_semantics=("parallel", "arbitrary"),
            vmem_limit_bytes=56 * 1024 * 1024,
        ),
    )(dbl_parts, dt_proj_w, dtb, alogt, ssmt, u, res, d2, out_proj_w)

    out = out_parts[0] + out_parts[1]
    return out.reshape(1, 1, B, DM)
